# Initial kernel scaffold; baseline (speedup 1.0000x reference)
import jax
import jax.numpy as jnp
from jax.experimental import pallas as pl


def kernel(x, edge_index, edge_feat, W1e, b1e, W2e, b2e, W1n, b1n, W2n, b2n):
    raise NotImplementedError("write your pallas kernel here")


# speedup vs baseline: 2.6880x; 2.6880x over previous
import functools

import jax
import jax.numpy as jnp
from jax import lax
from jax.experimental import pallas as pl
from jax.experimental.pallas import tpu as pltpu
from jax.experimental.pallas import tpu_sc as plsc

N = 10000
E = 320000
D = 128
F = 16
LANES = 16
NC, NS = 2, 16
NW = NC * NS
EPW = E // NW
CH = 80
NCH = EPW // CH
NPAD = 10240
NPT = NPAD // NS

_mesh = lambda: plsc.VectorSubcoreMesh(core_axis_name="c", subcore_axis_name="s")


def _pq_body(x_ref, wr_ref, wc_ref, p_ref, q_ref):
    xv = x_ref[...]
    p_ref[...] = jnp.dot(xv, wr_ref[...], preferred_element_type=jnp.float32)
    q_ref[...] = jnp.dot(xv, wc_ref[...], preferred_element_type=jnp.float32)


def _pq(x, wr, wc):
    return pl.pallas_call(
        _pq_body,
        out_shape=[jax.ShapeDtypeStruct((N, D), jnp.float32),
                   jax.ShapeDtypeStruct((N, D), jnp.float32)],
    )(x, wr, wc)


def _gather_body(p_hbm, q_hbm, row_hbm, col_hbm, out_hbm,
                 idxr, idxc, buf_p, buf_q, sem_p, sem_q):
    wid = lax.axis_index("s") * NC + lax.axis_index("c")
    base = wid * EPW

    def chunk(k, carry):
        off = base + k * CH
        pltpu.sync_copy(row_hbm.at[pl.ds(off, CH)], idxr)
        pltpu.sync_copy(col_hbm.at[pl.ds(off, CH)], idxc)
        cp_p = pltpu.async_copy(p_hbm.at[idxr], buf_p, sem_p)
        cp_q = pltpu.async_copy(q_hbm.at[idxc], buf_q, sem_q)
        cp_p.wait()
        cp_q.wait()

        def addrow(e, c2):
            for j in range(D // LANES):
                sl = pl.ds(j * LANES, LANES)
                buf_p[e, sl] = buf_p[e, sl] + buf_q[e, sl]
            return c2

        lax.fori_loop(0, CH, addrow, None)
        pltpu.sync_copy(buf_p, out_hbm.at[pl.ds(off, CH)])
        return carry

    lax.fori_loop(0, NCH, chunk, None)


def _gather(p, q, row, col):
    fn = pl.kernel(
        _gather_body,
        out_type=jax.ShapeDtypeStruct((E, D), jnp.float32),
        mesh=_mesh(),
        scratch_types=[
            pltpu.VMEM((CH,), jnp.int32),
            pltpu.VMEM((CH,), jnp.int32),
            pltpu.VMEM((CH, D), jnp.float32),
            pltpu.VMEM((CH, D), jnp.float32),
            pltpu.SemaphoreType.DMA,
            pltpu.SemaphoreType.DMA,
        ],
    )
    return fn(p, q, row, col)


_BE = 2000


def _emlp_body(g_ref, ef_ref, wf_ref, b1_ref, w2_ref, b2_ref, o_ref):
    pre = (g_ref[...]
           + jnp.dot(ef_ref[...], wf_ref[...], preferred_element_type=jnp.float32)
           + b1_ref[...])
    h = pre * jax.nn.sigmoid(pre)
    z = jnp.dot(h, w2_ref[...], preferred_element_type=jnp.float32) + b2_ref[...]
    o_ref[...] = z * jax.nn.sigmoid(z)


def _emlp(g, ef, wf, b1, w2, b2):
    return pl.pallas_call(
        _emlp_body,
        grid=(E // _BE,),
        in_specs=[
            pl.BlockSpec((_BE, D), lambda i: (i, 0)),
            pl.BlockSpec((_BE, F), lambda i: (i, 0)),
            pl.BlockSpec((F, D), lambda i: (0, 0)),
            pl.BlockSpec((1, D), lambda i: (0, 0)),
            pl.BlockSpec((D, D), lambda i: (0, 0)),
            pl.BlockSpec((1, D), lambda i: (0, 0)),
        ],
        out_specs=pl.BlockSpec((_BE, D), lambda i: (i, 0)),
        out_shape=jax.ShapeDtypeStruct((E, D), jnp.float32),
    )(g, ef, wf, b1, w2, b2)


def _scatter_body(m_hbm, row_hbm, zeros_hbm, out_hbm,
                  idxv, mbuf, acc, sem):
    c = lax.axis_index("c")
    s = lax.axis_index("s")
    pltpu.sync_copy(zeros_hbm, acc.at[pl.ds(s * NPT, NPT)])
    plsc.subcore_barrier()

    base = (c * NS + s) * EPW

    def chunk(k, carry):
        off = base + k * CH
        pltpu.sync_copy(row_hbm.at[pl.ds(off, CH)], idxv)
        pltpu.sync_copy(m_hbm.at[pl.ds(off, CH)], mbuf)
        pltpu.sync_copy(mbuf, acc.at[idxv], add=True)
        return carry

    lax.fori_loop(0, NCH, chunk, None)
    plsc.subcore_barrier()
    pltpu.sync_copy(acc.at[pl.ds(s * NPT, NPT)], out_hbm.at[c, pl.ds(s * NPT, NPT)])


def _scatter(m, row, zeros):
    fn = pl.kernel(
        _scatter_body,
        out_type=jax.ShapeDtypeStruct((NC, NPAD, D), jnp.float32),
        mesh=_mesh(),
        scratch_types=[
            pltpu.VMEM((CH,), jnp.int32),
            pltpu.VMEM((CH, D), jnp.float32),
            pltpu.VMEM_SHARED((NPAD, D), jnp.float32),
            pltpu.SemaphoreType.DMA,
        ],
    )
    return fn(m, row, zeros)


def _nmlp_body(x_ref, parts_ref, wx_ref, wa_ref, b1_ref, w2_ref, b2_ref, o_ref):
    agg = parts_ref[0] + parts_ref[1]
    pre = (jnp.dot(x_ref[...], wx_ref[...], preferred_element_type=jnp.float32)
           + jnp.dot(agg, wa_ref[...], preferred_element_type=jnp.float32)
           + b1_ref[...])
    h = pre * jax.nn.sigmoid(pre)
    o_ref[...] = jnp.dot(h, w2_ref[...], preferred_element_type=jnp.float32) + b2_ref[...]


def _nmlp(x, parts, wx, wa, b1, w2, b2):
    return pl.pallas_call(
        _nmlp_body,
        out_shape=jax.ShapeDtypeStruct((N, D), jnp.float32),
    )(x, parts, wx, wa, b1, w2, b2)


def kernel(x, edge_index, edge_feat, W1e, b1e, W2e, b2e, W1n, b1n, W2n, b2n):
    row = edge_index[0]
    col = edge_index[1]
    p, q = _pq(x, W1e[:D], W1e[D:2 * D])
    g = _gather(p, q, row, col)
    m = _emlp(g, edge_feat, W1e[2 * D:], b1e.reshape(1, D), W2e,
              b2e.reshape(1, D))
    parts = _scatter(m, row, jnp.zeros((NPT, D), jnp.float32))
    return _nmlp(x, parts[:, :N], W1n[:D], W1n[D:], b1n.reshape(1, D), W2n,
                 b2n.reshape(1, D))
